# Initial kernel scaffold; baseline (speedup 1.0000x reference)
import jax
import jax.numpy as jnp
from jax.experimental import pallas as pl


def kernel(query_tokens, document_tokens, embedding_table, score_w, score_b):
    raise NotImplementedError("write your pallas kernel here")


# speedup vs baseline: 3.6533x; 3.6533x over previous
import functools

import jax
import jax.numpy as jnp
from jax import lax
from jax.experimental import pallas as pl
from jax.experimental.pallas import tpu as pltpu
from jax.experimental.pallas import tpu_sc as plsc

NC, NS = 2, 16
NW = NC * NS
G = 128


def _make_sc_gather(V, D, N, per_w, gpc_rows, n_chunks):
    ch = gpc_rows * G

    mesh = plsc.VectorSubcoreMesh(core_axis_name="c", subcore_axis_name="s")

    rows_per_w = per_w // G

    @functools.partial(
        pl.kernel,
        out_type=jax.ShapeDtypeStruct((N, D), jnp.float32),
        mesh=mesh,
        scratch_types=[
            pltpu.VMEM((rows_per_w, G), jnp.int32),
            pltpu.VMEM((ch, D), jnp.float32),
            pltpu.SemaphoreType.DMA,
        ],
    )
    def sc_gather(table_hbm, idx_hbm, out_hbm, idx_v, rows_v, gsem):
        wid = lax.axis_index("s") * NC + lax.axis_index("c")
        out_row0 = wid * per_w

        pltpu.sync_copy(idx_hbm.at[wid], idx_v)

        def chunk(ci, carry):
            r0 = ci * gpc_rows
            copies = []
            for j in range(gpc_rows):
                copies.append(pltpu.async_copy(
                    table_hbm.at[idx_v.at[r0 + j]],
                    rows_v.at[pl.ds(j * G, G)],
                    gsem))
            for cp in copies:
                cp.wait()
            pltpu.sync_copy(rows_v, out_hbm.at[pl.ds(out_row0 + ci * ch, ch)])
            return carry

        lax.fori_loop(0, n_chunks, chunk, 0)

    return sc_gather


def _score_body(d_ref, q_ref, dt_ref, qt_ref, w_ref, b_ref, out_ref):
    BB, LD = dt_ref.shape
    _, LQ = qt_ref.shape
    D = d_ref.shape[1]
    d = d_ref[...]
    q = q_ref[...]
    dn = d / (jnp.sqrt(jnp.sum(d * d, axis=1, keepdims=True)) + 1e-10)
    qn = q / (jnp.sqrt(jnp.sum(q * q, axis=1, keepdims=True)) + 1e-10)
    w = w_ref[0, 0]
    b = b_ref[0, 0]
    for bi in range(BB):
        qb = lax.slice(qn, (bi * LQ, 0), ((bi + 1) * LQ, D))
        db = lax.slice(dn, (bi * LD, 0), ((bi + 1) * LD, D))
        sim = lax.dot_general(qb, db, (((1,), (1,)), ((), ())),
                              preferred_element_type=jnp.float32)
        dm = (dt_ref[bi, :] > 0).astype(jnp.float32)
        sim = sim * dm[None, :]
        mx = jnp.max(sim, axis=1)
        qm = (qt_ref[bi, :] > 0).astype(jnp.float32)
        pooled = jnp.sum(mx * qm)
        out_ref[bi, :] = jnp.full((D,), pooled * w + b, dtype=jnp.float32)


def kernel(query_tokens, document_tokens, embedding_table, score_w, score_b):
    B, LQ = query_tokens.shape
    _, LD = document_tokens.shape
    V, D = embedding_table.shape
    NQ, ND = B * LQ, B * LD
    N = NQ + ND

    per_w = N // NW
    assert per_w * NW == N and per_w % G == 0
    gpc_rows = 5
    n_chunks = per_w // (gpc_rows * G)
    assert n_chunks * gpc_rows * G == per_w

    idx = jnp.concatenate(
        [document_tokens.reshape(-1), query_tokens.reshape(-1)]
    ).astype(jnp.int32)
    idx3d = idx.reshape(NW, per_w // G, G)

    sc_gather = _make_sc_gather(V, D, N, per_w, gpc_rows, n_chunks)
    gathered = sc_gather(embedding_table.astype(jnp.float32), idx3d)

    BB = 8
    grid = (B // BB,)
    q_blk0 = ND // (BB * LQ)
    out = pl.pallas_call(
        _score_body,
        grid=grid,
        in_specs=[
            pl.BlockSpec((BB * LD, D), lambda i: (i, 0)),
            pl.BlockSpec((BB * LQ, D), lambda i: (i + q_blk0, 0)),
            pl.BlockSpec((BB, LD), lambda i: (i, 0)),
            pl.BlockSpec((BB, LQ), lambda i: (i, 0)),
            pl.BlockSpec(memory_space=pltpu.SMEM),
            pl.BlockSpec(memory_space=pltpu.SMEM),
        ],
        out_specs=pl.BlockSpec((BB, D), lambda i: (i, 0)),
        out_shape=jax.ShapeDtypeStruct((B, D), jnp.float32),
        compiler_params=pltpu.CompilerParams(
            dimension_semantics=("parallel",)),
    )(gathered, gathered,
      document_tokens.astype(jnp.int32), query_tokens.astype(jnp.int32),
      jnp.reshape(score_w, (1, 1)).astype(jnp.float32),
      jnp.reshape(score_b, (1, 1)).astype(jnp.float32))
    return out[:, 0]
